# baseline (speedup 1.0000x reference)
import functools

import jax
import jax.numpy as jnp
from jax import lax
from jax.experimental import pallas as pl
from jax.experimental.pallas import tpu as pltpu
from jax.experimental.pallas import tpu_sc as plsc

B = 1024
V = 100000

_info = plsc.get_sparse_core_info()
_NC, _NS, _L = _info.num_cores, _info.num_subcores, _info.num_lanes
_NW = _NC * _NS
_BPW = B // _NW

_mesh = plsc.VectorSubcoreMesh(core_axis_name="c", subcore_axis_name="s")


@functools.partial(
    pl.kernel,
    mesh=_mesh,
    out_type=jax.ShapeDtypeStruct((B,), jnp.float32),
    scratch_types=[
        pltpu.VMEM((_BPW,), jnp.int32),
        pltpu.VMEM((_BPW, 128), jnp.float32),
        pltpu.VMEM((_BPW,), jnp.float32),
        pltpu.SemaphoreType.DMA,
    ],
    compiler_params=pltpu.CompilerParams(
        use_tc_tiling_on_sc=True,
        needs_layout_passes=False,
        skip_device_barrier=True,
    ),
)
def _gather_loss(in_hbm, tgt_hbm, out_hbm, tgt_v, rows_v, val_v, sem):
    wid = lax.axis_index("s") * _NC + lax.axis_index("c")
    base = wid * _BPW
    pltpu.sync_copy(tgt_hbm.at[pl.ds(base, _BPW)], tgt_v)
    iota = lax.broadcasted_iota(jnp.int32, (_L,), 0)
    copies = []
    for k in range(_BPW):
        chunk = tgt_v[pl.ds((k // _L) * _L, _L)]
        t = jnp.sum(jnp.where(iota == (k % _L), chunk, 0))
        c0 = pl.multiple_of((t // 128) * 128, 128)
        cp = pltpu.make_async_copy(
            in_hbm.at[pl.ds(base + k, 1), pl.ds(c0, 128)],
            rows_v.at[pl.ds(k, 1), :],
            sem,
        )
        cp.start()
        copies.append(cp)
    for cp in copies:
        cp.wait()
    for j in range(_BPW // _L):
        ridx = lax.broadcasted_iota(jnp.int32, (_L,), 0) + j * _L
        cidx = tgt_v[pl.ds(j * _L, _L)] % 128
        val_v[pl.ds(j * _L, _L)] = plsc.load_gather(rows_v, [ridx, cidx])
    pltpu.sync_copy(val_v, out_hbm.at[pl.ds(base, _BPW)])


def kernel(inputs, targets):
    tgt = targets.astype(jnp.int32)
    return _gather_loss(inputs, tgt)


# speedup vs baseline: 1.0013x; 1.0013x over previous
import functools

import jax
import jax.numpy as jnp
from jax import lax
from jax.experimental import pallas as pl
from jax.experimental.pallas import tpu as pltpu
from jax.experimental.pallas import tpu_sc as plsc

B = 1024
V = 100000

_info = plsc.get_sparse_core_info()
_NC, _NS, _L = _info.num_cores, _info.num_subcores, _info.num_lanes
_NW = _NC * _NS
_BPW = B // _NW

_mesh = plsc.VectorSubcoreMesh(core_axis_name="c", subcore_axis_name="s")


@functools.partial(
    pl.kernel,
    mesh=_mesh,
    out_type=jax.ShapeDtypeStruct((B,), jnp.float32),
    scratch_types=[
        pltpu.VMEM((_BPW,), jnp.int32),
        pltpu.VMEM((_BPW, 128), jnp.float32),
        pltpu.VMEM((_BPW,), jnp.float32),
        pltpu.SemaphoreType.DMA,
    ],
    compiler_params=pltpu.CompilerParams(
        use_tc_tiling_on_sc=True,
        needs_layout_passes=False,
    ),
)
def _gather_loss(in_hbm, tgt_hbm, out_hbm, tgt_v, rows_v, val_v, sem):
    wid = lax.axis_index("s") * _NC + lax.axis_index("c")
    base = wid * _BPW
    pltpu.sync_copy(tgt_hbm.at[pl.ds(base, _BPW)], tgt_v)
    iota = lax.broadcasted_iota(jnp.int32, (_L,), 0)
    copies = []
    for k in range(_BPW):
        chunk = tgt_v[pl.ds((k // _L) * _L, _L)]
        t = jnp.sum(jnp.where(iota == (k % _L), chunk, 0))
        c0 = pl.multiple_of((t // 128) * 128, 128)
        cp = pltpu.make_async_copy(
            in_hbm.at[pl.ds(base + k, 1), pl.ds(c0, 128)],
            rows_v.at[pl.ds(k, 1), :],
            sem,
        )
        cp.start()
        copies.append(cp)
    for cp in copies:
        cp.wait()
    for j in range(_BPW // _L):
        ridx = lax.broadcasted_iota(jnp.int32, (_L,), 0) + j * _L
        cidx = tgt_v[pl.ds(j * _L, _L)] % 128
        val_v[pl.ds(j * _L, _L)] = plsc.load_gather(rows_v, [ridx, cidx])
    pltpu.sync_copy(val_v, out_hbm.at[pl.ds(base, _BPW)])


def kernel(inputs, targets):
    tgt = targets.astype(jnp.int32)
    return _gather_loss(inputs, tgt)
